# baseline (speedup 1.0000x reference)
import functools

import jax
import jax.numpy as jnp
from jax import lax
from jax.experimental import pallas as pl
from jax.experimental.pallas import tpu as pltpu
from jax.experimental.pallas import tpu_sc as plsc

_B, _C, _N, _K = 1, 128, 10000, 32
_OUT = 128
_NTILES = 32
_CPT = _C // _NTILES
_CHUNK = 400
_NGROUPS = _CHUNK // 16
_NCHUNKS = _N // _CHUNK
_NEG = -1e30


def _sc_max_relative(xt, e0t, e1t):
    mesh = plsc.VectorSubcoreMesh(core_axis_name="c", subcore_axis_name="s")

    @functools.partial(
        pl.kernel,
        out_type=jax.ShapeDtypeStruct((_C, _N), jnp.float32),
        mesh=mesh,
        scratch_types=[
            pltpu.VMEM((_CPT, _N), jnp.float32),
            pltpu.VMEM((2, _K, _CHUNK), jnp.int32),
            pltpu.VMEM((_CPT, _CHUNK), jnp.float32),
        ],
        compiler_params=pltpu.CompilerParams(
            use_tc_tiling_on_sc=False, needs_layout_passes=False),
    )
    def sc_kernel(xt_hbm, e0_hbm, e1_hbm, out_hbm, xt_v, idx_v, out_v):
        wid = lax.axis_index("s") * 2 + lax.axis_index("c")
        c0 = wid * _CPT
        pltpu.sync_copy(xt_hbm.at[pl.ds(c0, _CPT), :], xt_v)

        def chunk_body(ci, carry):
            col = ci * _CHUNK
            pltpu.sync_copy(e0_hbm.at[:, pl.ds(col, _CHUNK)], idx_v.at[0])
            pltpu.sync_copy(e1_hbm.at[:, pl.ds(col, _CHUNK)], idx_v.at[1])

            def group_body(g, gcarry):
                base = g * 16
                accs = [jnp.full((16,), _NEG, jnp.float32) for _ in range(_CPT)]
                for kk in range(_K):
                    i0 = idx_v[0, kk, pl.ds(base, 16)]
                    i1 = idx_v[1, kk, pl.ds(base, 16)]
                    valid = i0 != i1
                    for c in range(_CPT):
                        xj = plsc.load_gather(xt_v.at[c], [i0])
                        xi = plsc.load_gather(xt_v.at[c], [i1])
                        d = jnp.where(valid, xj - xi, _NEG)
                        accs[c] = jnp.maximum(accs[c], d)
                for c in range(_CPT):
                    out_v[c, pl.ds(base, 16)] = accs[c]
                return gcarry

            lax.fori_loop(0, _NGROUPS, group_body, 0)
            pltpu.sync_copy(out_v, out_hbm.at[pl.ds(c0, _CPT), pl.ds(col, _CHUNK)])
            return carry

        lax.fori_loop(0, _NCHUNKS, chunk_body, 0)

    return sc_kernel(xt, e0t, e1t)


_BN = 1024


def _tc_conv(xt, m, W, b2):
    def body(w_ref, b_ref, x_ref, m_ref, o_ref):
        acc = jnp.dot(w_ref[:, :_C], x_ref[...],
                      preferred_element_type=jnp.float32)
        acc = acc + jnp.dot(w_ref[:, _C:], m_ref[...],
                            preferred_element_type=jnp.float32)
        o_ref[...] = jnp.maximum(acc + b_ref[...], 0.0)

    grid = (pl.cdiv(_N, _BN),)
    return pl.pallas_call(
        body,
        grid=grid,
        in_specs=[
            pl.BlockSpec((_OUT, 2 * _C), lambda i: (0, 0)),
            pl.BlockSpec((_OUT, 1), lambda i: (0, 0)),
            pl.BlockSpec((_C, _BN), lambda i: (0, i)),
            pl.BlockSpec((_C, _BN), lambda i: (0, i)),
        ],
        out_specs=pl.BlockSpec((_OUT, _BN), lambda i: (0, i)),
        out_shape=jax.ShapeDtypeStruct((_OUT, _N), jnp.float32),
    )(W, b2, xt, m)


def kernel(x, x_0, W, b, edge_index):
    xt = x[0, :, :, 0]
    e = edge_index.astype(jnp.int32)
    e0t = jnp.transpose(e[0, 0])
    e1t = jnp.transpose(e[1, 0])
    m = _sc_max_relative(xt, e0t, e1t)
    out = _tc_conv(xt, m, W, b.reshape(_OUT, 1))
    return out[None, :, :, None]


# speedup vs baseline: 1.5717x; 1.5717x over previous
import functools

import jax
import jax.numpy as jnp
from jax import lax
from jax.experimental import pallas as pl
from jax.experimental.pallas import tpu as pltpu
from jax.experimental.pallas import tpu_sc as plsc

_B, _C, _N, _K = 1, 128, 10000, 32
_OUT = 128
_NTILES = 32
_CPT = _C // _NTILES
_CHUNK = 400
_NGROUPS = _CHUNK // 16
_NCHUNKS = _N // _CHUNK
_NEG = -1e30


_POS_PAIR = 0x7F7F7F7F
_NEG_PAIR = 0xFF7FFF7F - 0x100000000
_CPP = _CPT // 2


def _sc_max_relative(xp, e0t, e1t):
    mesh = plsc.VectorSubcoreMesh(core_axis_name="c", subcore_axis_name="s")

    @functools.partial(
        pl.kernel,
        out_type=jax.ShapeDtypeStruct((_C, _N), jnp.float32),
        mesh=mesh,
        scratch_types=[
            pltpu.VMEM((_CPP, _N), jnp.int32),
            pltpu.VMEM((2, 2, _K, _CHUNK), jnp.int32),
            pltpu.VMEM((_CPT, _CHUNK), jnp.float32),
            pltpu.SemaphoreType.DMA((2,)),
        ],
        compiler_params=pltpu.CompilerParams(
            use_tc_tiling_on_sc=False, needs_layout_passes=False),
    )
    def sc_kernel(xp_hbm, e0_hbm, e1_hbm, out_hbm, xt_v, idx_v, out_v, sems):
        wid = lax.axis_index("s") * 2 + lax.axis_index("c")
        c0 = wid * _CPT
        pltpu.sync_copy(xp_hbm.at[pl.ds(wid * _CPP, _CPP), :], xt_v)

        def start_fetch(ci, buf):
            col = ci * _CHUNK
            pltpu.async_copy(
                e0_hbm.at[:, pl.ds(col, _CHUNK)], idx_v.at[buf, 0], sems.at[buf])
            pltpu.async_copy(
                e1_hbm.at[:, pl.ds(col, _CHUNK)], idx_v.at[buf, 1], sems.at[buf])

        def wait_fetch(buf):
            pltpu.make_async_copy(
                e0_hbm.at[:, pl.ds(0, _CHUNK)], idx_v.at[buf, 0], sems.at[buf]
            ).wait()
            pltpu.make_async_copy(
                e1_hbm.at[:, pl.ds(0, _CHUNK)], idx_v.at[buf, 1], sems.at[buf]
            ).wait()

        start_fetch(0, 0)

        def chunk_compute(ci, buf):
            col = ci * _CHUNK

            def group_body(g, gcarry):
                base = g * 16
                neg = plsc.bitcast(
                    jnp.full((16,), _NEG_PAIR, jnp.int32), jnp.bfloat16)
                accs = [neg for _ in range(_CPP)]
                for kk in range(_K):
                    i0 = idx_v[buf, 0, kk, pl.ds(base, 16)]
                    i1 = idx_v[buf, 1, kk, pl.ds(base, 16)]
                    valid = i0 != i1
                    cap = plsc.bitcast(
                        jnp.where(valid, jnp.int32(_POS_PAIR),
                                  jnp.int32(_NEG_PAIR)),
                        jnp.bfloat16)
                    for cp in range(_CPP):
                        xj = plsc.bitcast(
                            plsc.load_gather(xt_v.at[cp], [i0]), jnp.bfloat16)
                        xi = plsc.bitcast(
                            plsc.load_gather(xt_v.at[cp], [i1]), jnp.bfloat16)
                        d = jnp.minimum(xj - xi, cap)
                        accs[cp] = jnp.maximum(accs[cp], d)
                for cp in range(_CPP):
                    a, b = plsc.unpack(
                        accs[cp], format=plsc.PackFormat.INTERLEAVED)
                    out_v[2 * cp, pl.ds(base, 16)] = jnp.maximum(a, _NEG)
                    out_v[2 * cp + 1, pl.ds(base, 16)] = jnp.maximum(b, _NEG)
                return gcarry

            lax.fori_loop(0, _NGROUPS, group_body, 0)
            pltpu.sync_copy(out_v, out_hbm.at[pl.ds(c0, _CPT), pl.ds(col, _CHUNK)])

        def pair_body(p, carry):
            for b in range(2):
                ci = 2 * p + b

                @pl.when(ci < _NCHUNKS)
                def _():
                    @pl.when(ci + 1 < _NCHUNKS)
                    def _():
                        start_fetch(ci + 1, 1 - b)

                    wait_fetch(b)
                    chunk_compute(ci, b)

            return carry

        lax.fori_loop(0, (_NCHUNKS + 1) // 2, pair_body, 0)

    return sc_kernel(xp, e0t, e1t)


def _pack_pairs(xt):
    bits = lax.bitcast_convert_type(
        xt.astype(jnp.bfloat16), jnp.uint16).astype(jnp.uint32)
    packed = bits[0::2] | (bits[1::2] << 16)
    return lax.bitcast_convert_type(packed, jnp.int32)


_BN = 1024


def _tc_conv(xt, m, W, b2):
    def body(w_ref, b_ref, x_ref, m_ref, o_ref):
        acc = jnp.dot(w_ref[:, :_C], x_ref[...],
                      preferred_element_type=jnp.float32)
        acc = acc + jnp.dot(w_ref[:, _C:], m_ref[...],
                            preferred_element_type=jnp.float32)
        o_ref[...] = jnp.maximum(acc + b_ref[...], 0.0)

    grid = (pl.cdiv(_N, _BN),)
    return pl.pallas_call(
        body,
        grid=grid,
        in_specs=[
            pl.BlockSpec((_OUT, 2 * _C), lambda i: (0, 0)),
            pl.BlockSpec((_OUT, 1), lambda i: (0, 0)),
            pl.BlockSpec((_C, _BN), lambda i: (0, i)),
            pl.BlockSpec((_C, _BN), lambda i: (0, i)),
        ],
        out_specs=pl.BlockSpec((_OUT, _BN), lambda i: (0, i)),
        out_shape=jax.ShapeDtypeStruct((_OUT, _N), jnp.float32),
    )(W, b2, xt, m)


def kernel(x, x_0, W, b, edge_index):
    xt = x[0, :, :, 0]
    e = edge_index.astype(jnp.int32)
    e0t = jnp.transpose(e[0, 0])
    e1t = jnp.transpose(e[1, 0])
    m = _sc_max_relative(_pack_pairs(xt), e0t, e1t)
    out = _tc_conv(xt, m, W, b.reshape(_OUT, 1))
    return out[None, :, :, None]


# speedup vs baseline: 2.1190x; 1.3482x over previous
import functools

import jax
import jax.numpy as jnp
from jax import lax
from jax.experimental import pallas as pl
from jax.experimental.pallas import tpu as pltpu
from jax.experimental.pallas import tpu_sc as plsc

_B, _C, _N, _K = 1, 128, 10000, 32
_OUT = 128
_NTILES = 32
_CPT = _C // _NTILES
_CHUNK = 400
_NGROUPS = _CHUNK // 16
_NCHUNKS = _N // _CHUNK
_NEG = -1e30


_POS_PAIR = 0x7F7F7F7F
_NEG_PAIR = 0xFF7FFF7F - 0x100000000
_CPP = _CPT // 2


def _sc_max_relative(xp, et):
    mesh = plsc.VectorSubcoreMesh(core_axis_name="c", subcore_axis_name="s")

    @functools.partial(
        pl.kernel,
        out_type=jax.ShapeDtypeStruct((_C // 2, _N), jnp.int32),
        mesh=mesh,
        scratch_types=[
            pltpu.VMEM((_CPP, _N), jnp.int32),
            pltpu.VMEM((2, 2, _K, _CHUNK), jnp.int32),
            pltpu.VMEM((2, _CPP, _CHUNK), jnp.int32),
            pltpu.SemaphoreType.DMA((2,)),
            pltpu.SemaphoreType.DMA((2,)),
        ],
        compiler_params=pltpu.CompilerParams(
            use_tc_tiling_on_sc=False, needs_layout_passes=False),
    )
    def sc_kernel(xp_hbm, et_hbm, out_hbm, xt_v, idx_v, out_v, sems,
                  osems):
        wid = lax.axis_index("s") * 2 + lax.axis_index("c")
        r0 = wid * _CPP

        def start_fetch(ci, buf):
            col = ci * _CHUNK
            pltpu.async_copy(
                et_hbm.at[0, :, pl.ds(col, _CHUNK)], idx_v.at[buf, 0],
                sems.at[buf])
            pltpu.async_copy(
                et_hbm.at[1, :, pl.ds(col, _CHUNK)], idx_v.at[buf, 1],
                sems.at[buf])

        def wait_fetch(buf):
            for h in range(2):
                pltpu.make_async_copy(
                    et_hbm.at[h, :, pl.ds(0, _CHUNK)], idx_v.at[buf, h],
                    sems.at[buf],
                ).wait()

        start_fetch(0, 0)
        pltpu.sync_copy(xp_hbm.at[pl.ds(r0, _CPP), :], xt_v)

        def wait_out(buf):
            pltpu.make_async_copy(
                out_v.at[buf],
                out_hbm.at[pl.ds(0, _CPP), pl.ds(0, _CHUNK)],
                osems.at[buf],
            ).wait()

        def chunk_compute(ci, buf):
            col = ci * _CHUNK

            @pl.when(ci >= 2)
            def _():
                wait_out(buf)

            @plsc.parallel_loop(0, _NGROUPS)
            def group_body(g):
                base = g * 16
                neg = plsc.bitcast(
                    jnp.full((16,), _NEG_PAIR, jnp.int32), jnp.bfloat16)

                def k_body(k8, accs):
                    acc0, acc1 = accs
                    for kk8 in range(8):
                        kk = k8 * 8 + kk8
                        i0 = idx_v[buf, 0, kk, pl.ds(base, 16)]
                        i1 = idx_v[buf, 1, kk, pl.ds(base, 16)]
                        valid = i0 != i1
                        cap = plsc.bitcast(
                            jnp.where(valid, jnp.int32(_POS_PAIR),
                                      jnp.int32(_NEG_PAIR)),
                            jnp.bfloat16)
                        for cp in range(_CPP):
                            xj = plsc.bitcast(
                                plsc.load_gather(xt_v.at[cp], [i0]),
                                jnp.bfloat16)
                            xi = plsc.bitcast(
                                plsc.load_gather(xt_v.at[cp], [i1]),
                                jnp.bfloat16)
                            d = jnp.minimum(xj - xi, cap)
                            if cp == 0:
                                acc0 = jnp.maximum(acc0, d)
                            else:
                                acc1 = jnp.maximum(acc1, d)
                    return acc0, acc1

                accs = lax.fori_loop(0, _K // 8, k_body, (neg, neg))
                for cp in range(_CPP):
                    out_v[buf, cp, pl.ds(base, 16)] = plsc.bitcast(
                        accs[cp], jnp.int32)

            pltpu.async_copy(
                out_v.at[buf],
                out_hbm.at[pl.ds(r0, _CPP), pl.ds(col, _CHUNK)],
                osems.at[buf])

        def pair_body(p, carry):
            for b in range(2):
                ci = 2 * p + b

                @pl.when(ci < _NCHUNKS)
                def _():
                    @pl.when(ci + 1 < _NCHUNKS)
                    def _():
                        start_fetch(ci + 1, 1 - b)

                    wait_fetch(b)
                    chunk_compute(ci, b)

            return carry

        lax.fori_loop(0, (_NCHUNKS + 1) // 2, pair_body, 0)
        wait_out(0)
        wait_out(1)

    return sc_kernel(xp, et)


_BN = 2048


def _pack_halves(xt):
    bits = lax.bitcast_convert_type(
        xt.astype(jnp.bfloat16), jnp.uint16).astype(jnp.uint32)
    packed = bits[: _C // 2] | (bits[_C // 2:] << 16)
    return lax.bitcast_convert_type(packed, jnp.int32)


def _tc_xconv(xt, W1, b2):
    def body(w_ref, b_ref, x_ref, o_ref):
        o_ref[...] = jnp.dot(w_ref[...], x_ref[...],
                             preferred_element_type=jnp.float32) + b_ref[...]

    return pl.pallas_call(
        body,
        grid=(pl.cdiv(_N, _BN),),
        in_specs=[
            pl.BlockSpec((_OUT, _C), lambda i: (0, 0)),
            pl.BlockSpec((_OUT, 1), lambda i: (0, 0)),
            pl.BlockSpec((_C, _BN), lambda i: (0, i)),
        ],
        out_specs=pl.BlockSpec((_OUT, _BN), lambda i: (0, i)),
        out_shape=jax.ShapeDtypeStruct((_OUT, _N), jnp.float32),
    )(W1, b2, xt)


def _tc_mconv(p1, mp, W2):
    def body(w_ref, p_ref, m_ref, o_ref):
        words = m_ref[...]
        lo = lax.bitcast_convert_type(words << 16, jnp.float32)
        hi = lax.bitcast_convert_type(
            words & jnp.int32(-65536), jnp.float32)
        m_full = jnp.maximum(jnp.concatenate([lo, hi], axis=0), _NEG)
        acc = jnp.dot(w_ref[...], m_full,
                      preferred_element_type=jnp.float32)
        o_ref[...] = jnp.maximum(acc + p_ref[...], 0.0)

    return pl.pallas_call(
        body,
        grid=(pl.cdiv(_N, _BN),),
        in_specs=[
            pl.BlockSpec((_OUT, _C), lambda i: (0, 0)),
            pl.BlockSpec((_OUT, _BN), lambda i: (0, i)),
            pl.BlockSpec((_C // 2, _BN), lambda i: (0, i)),
        ],
        out_specs=pl.BlockSpec((_OUT, _BN), lambda i: (0, i)),
        out_shape=jax.ShapeDtypeStruct((_OUT, _N), jnp.float32),
    )(W2, p1, mp)


def kernel(x, x_0, W, b, edge_index):
    xt = x[0, :, :, 0]
    e2 = edge_index.astype(jnp.int32).reshape(2, _N, _K)
    et = jnp.transpose(e2, (0, 2, 1))
    m = _sc_max_relative(_pack_halves(xt), et)
    p1 = _tc_xconv(xt, W[:, :_C], b.reshape(_OUT, 1))
    out = _tc_mconv(p1, m, W[:, _C:])
    return out[None, :, :, None]
